# baseline (speedup 1.0000x reference)
import functools

import jax
import jax.numpy as jnp
from jax import lax
from jax.experimental import pallas as pl
from jax.experimental.pallas import tpu as pltpu
from jax.experimental.pallas import tpu_sc as plsc

EMBED_DIM = 64
NUM_WORKERS = 32
CHUNK = 125
GATHER_W = 128
NBUF = 5


@functools.partial(jax.jit, static_argnames=("n_rows",))
def _embed_gather(idx2d, table, *, n_rows):
    per_w = n_rows // NUM_WORKERS
    cpw = per_w // CHUNK
    groups = cpw // NBUF
    mesh = plsc.VectorSubcoreMesh(core_axis_name="c", subcore_axis_name="s")

    @functools.partial(
        pl.kernel,
        mesh=mesh,
        compiler_params=pltpu.CompilerParams(use_tc_tiling_on_sc=False),
        out_type=jax.ShapeDtypeStruct((n_rows, EMBED_DIM), jnp.float32),
        scratch_types=[
            pltpu.VMEM((cpw, GATHER_W), jnp.int32),
            pltpu.VMEM((NBUF, GATHER_W, EMBED_DIM), jnp.float32),
            pltpu.SemaphoreType.DMA((NBUF,)),
            pltpu.SemaphoreType.DMA((NBUF,)),
        ],
    )
    def k(idx_hbm, table_hbm, out_hbm, idx_v, rows_v, gsem, osem):
        wid = lax.axis_index("s") * 2 + lax.axis_index("c")
        base = wid * per_w
        pltpu.sync_copy(idx_hbm.at[pl.ds(wid * cpw, cpw)], idx_v)

        def gather_start(j, b):
            pltpu.async_copy(
                table_hbm.at[idx_v.at[j]], rows_v.at[b], gsem.at[b]
            )

        def gather_wait(b):
            pltpu.make_async_copy(
                table_hbm.at[pl.ds(0, GATHER_W)], rows_v.at[b], gsem.at[b]
            ).wait()

        def out_start(j, b):
            pltpu.async_copy(
                rows_v.at[b, pl.ds(0, CHUNK)],
                out_hbm.at[pl.ds(base + j * CHUNK, CHUNK)],
                osem.at[b],
            )

        def out_wait(b):
            pltpu.make_async_copy(
                rows_v.at[b, pl.ds(0, CHUNK)],
                out_hbm.at[pl.ds(base, CHUNK)],
                osem.at[b],
            ).wait()

        for b in range(NBUF):
            gather_start(b, b)

        @pl.loop(0, groups - 1)
        def _(g):
            for t in range(NBUF):
                j = g * NBUF + t
                gather_wait(t)
                out_start(j, t)
                out_wait(t)
                gather_start(j + NBUF, t)

        for t in range(NBUF):
            j = (groups - 1) * NBUF + t
            gather_wait(t)
            out_start(j, t)
        for t in range(NBUF):
            out_wait(t)

    return k(idx2d, table)


def kernel(tokens, emb_weight):
    n = tokens.shape[0]
    per_w = n // NUM_WORKERS
    assert per_w * NUM_WORKERS == n and per_w % CHUNK == 0
    idx = tokens.astype(jnp.int32).reshape(n // CHUNK, CHUNK)
    idx2d = jnp.pad(idx, ((0, 0), (0, GATHER_W - CHUNK)))
    return _embed_gather(idx2d, emb_weight, n_rows=n)


# speedup vs baseline: 1.0327x; 1.0327x over previous
import functools

import jax
import jax.numpy as jnp
from jax import lax
from jax.experimental import pallas as pl
from jax.experimental.pallas import tpu as pltpu
from jax.experimental.pallas import tpu_sc as plsc

EMBED_DIM = 64
NUM_WORKERS = 32
CHUNK = 128
NBUF = 5


@functools.partial(jax.jit, static_argnames=("n_rows",))
def _embed_gather(idx2d, table, *, n_rows):
    per_w = n_rows // NUM_WORKERS
    cpw = idx2d.shape[0] // NUM_WORKERS
    tail = per_w - (cpw - 1) * CHUNK
    groups = cpw // NBUF
    mesh = plsc.VectorSubcoreMesh(core_axis_name="c", subcore_axis_name="s")

    @functools.partial(
        pl.kernel,
        mesh=mesh,
        compiler_params=pltpu.CompilerParams(use_tc_tiling_on_sc=False),
        out_type=jax.ShapeDtypeStruct((n_rows, EMBED_DIM), jnp.float32),
        scratch_types=[
            pltpu.VMEM((cpw, CHUNK), jnp.int32),
            pltpu.VMEM((NBUF, CHUNK, EMBED_DIM), jnp.float32),
            pltpu.SemaphoreType.DMA((NBUF,)),
            pltpu.SemaphoreType.DMA((NBUF,)),
        ],
    )
    def k(idx_hbm, table_hbm, out_hbm, idx_v, rows_v, gsem, osem):
        wid = lax.axis_index("s") * 2 + lax.axis_index("c")
        base = wid * per_w
        pltpu.sync_copy(idx_hbm.at[pl.ds(wid * cpw, cpw)], idx_v)

        def gather_start(j, b):
            pltpu.async_copy(
                table_hbm.at[idx_v.at[j]], rows_v.at[b], gsem.at[b]
            )

        def gather_wait(b):
            pltpu.make_async_copy(
                table_hbm.at[pl.ds(0, CHUNK)], rows_v.at[b], gsem.at[b]
            ).wait()

        def out_start(j, b, width):
            pltpu.async_copy(
                rows_v.at[b, pl.ds(0, width)],
                out_hbm.at[pl.ds(base + j * CHUNK, width)],
                osem.at[b],
            )

        def out_wait(b, width):
            pltpu.make_async_copy(
                rows_v.at[b, pl.ds(0, width)],
                out_hbm.at[pl.ds(base, width)],
                osem.at[b],
            ).wait()

        for b in range(NBUF):
            gather_start(b, b)

        @pl.loop(0, groups - 1)
        def _(g):
            for t in range(NBUF):
                j = g * NBUF + t
                gather_wait(t)
                out_start(j, t, CHUNK)
                out_wait(t, CHUNK)
                gather_start(j + NBUF, t)

        for t in range(NBUF):
            j = (groups - 1) * NBUF + t
            width = CHUNK if t < NBUF - 1 else tail
            gather_wait(t)
            out_start(j, t, width)
        for t in range(NBUF):
            width = CHUNK if t < NBUF - 1 else tail
            out_wait(t, width)

    return k(idx2d, table)


def kernel(tokens, emb_weight):
    n = tokens.shape[0]
    per_w = n // NUM_WORKERS
    assert per_w * NUM_WORKERS == n
    cpw = -(-per_w // CHUNK)
    idx = tokens.astype(jnp.int32).reshape(NUM_WORKERS, per_w)
    idx = jnp.pad(idx, ((0, 0), (0, cpw * CHUNK - per_w)))
    idx2d = idx.reshape(NUM_WORKERS * cpw, CHUNK)
    return _embed_gather(idx2d, emb_weight, n_rows=n)
